# Initial kernel scaffold; baseline (speedup 1.0000x reference)
import functools

import jax
import jax.numpy as jnp
from jax import lax
from jax.experimental import pallas as pl
from jax.experimental.pallas import tpu as pltpu
from jax.experimental.pallas import tpu_sc as plsc

R, C = 16384, 200
V, D = 100, 36
N = R * C
G = 128
NG = N // G
NC, NS = 2, 16
NW = NC * NS
GPW = NG // NW
K = 8
NCHUNK = GPW // K


def _sc_gather(x2d, wte):
    mesh = plsc.VectorSubcoreMesh(core_axis_name="c", subcore_axis_name="s")

    @functools.partial(
        pl.kernel,
        mesh=mesh,
        out_type=jax.ShapeDtypeStruct((NG, G, D), jnp.float32),
        scratch_types=[
            pltpu.VMEM((K, G), jnp.int32),
            pltpu.VMEM((K, G, D), jnp.float32),
            pltpu.SemaphoreType.DMA,
        ],
    )
    def k(x_hbm, wte_hbm, out_hbm, idx_v, rows_v, sem):
        wid = lax.axis_index("s") * NC + lax.axis_index("c")
        base = wid * GPW

        def chunk(i, carry):
            g0 = base + i * K
            pltpu.sync_copy(x_hbm.at[pl.ds(g0, K)], idx_v)
            copies = [
                pltpu.async_copy(wte_hbm.at[idx_v.at[j]], rows_v.at[j], sem)
                for j in range(K)
            ]
            for c in copies:
                c.wait()
            pltpu.sync_copy(rows_v, out_hbm.at[pl.ds(g0, K)])
            return carry

        lax.fori_loop(0, NCHUNK, chunk, 0)

    return k(x2d, wte)


def kernel(x, wte):
    x2d = x.reshape(NG, G).astype(jnp.int32)
    out = _sc_gather(x2d, wte)
    return out.reshape(R, C, D)


# speedup vs baseline: 3.0797x; 3.0797x over previous
import functools

import jax
import jax.numpy as jnp
from jax import lax
from jax.experimental import pallas as pl
from jax.experimental.pallas import tpu as pltpu
from jax.experimental.pallas import tpu_sc as plsc

R, C = 16384, 200
V, D = 100, 36
D2 = 2 * D
NP = R * C // 2
G = 128
NG = NP // G
NC, NS = 2, 16
NW = NC * NS
GPW = NG // NW
K = 8
NCHUNK = GPW // K


def _sc_gather(xp2d, ptab):
    mesh = plsc.VectorSubcoreMesh(core_axis_name="c", subcore_axis_name="s")

    @functools.partial(
        pl.kernel,
        mesh=mesh,
        out_type=jax.ShapeDtypeStruct((NG, G, D2), jnp.float32),
        scratch_types=[
            pltpu.VMEM((K, G), jnp.int32),
            pltpu.VMEM((K, G, D2), jnp.float32),
            pltpu.SemaphoreType.DMA,
        ],
        compiler_params=pltpu.CompilerParams(use_tc_tiling_on_sc=False),
    )
    def k(xp_hbm, p_hbm, out_hbm, idx_v, rows_v, sem):
        wid = lax.axis_index("s") * NC + lax.axis_index("c")
        base = wid * GPW

        def chunk(i, carry):
            g0 = base + i * K
            pltpu.sync_copy(xp_hbm.at[pl.ds(g0, K)], idx_v)
            copies = [
                pltpu.async_copy(p_hbm.at[idx_v.at[j]], rows_v.at[j], sem)
                for j in range(K)
            ]
            for c in copies:
                c.wait()
            pltpu.sync_copy(rows_v, out_hbm.at[pl.ds(g0, K)])
            return carry

        lax.fori_loop(0, NCHUNK, chunk, 0)

    return k(xp2d, ptab)


def kernel(x, wte):
    xr = x.reshape(NP, 2).astype(jnp.int32)
    xp2d = (xr[:, 0] * V + xr[:, 1]).reshape(NG, G)
    ptab = jnp.concatenate(
        [
            jnp.broadcast_to(wte[:, None, :], (V, V, D)),
            jnp.broadcast_to(wte[None, :, :], (V, V, D)),
        ],
        axis=-1,
    ).reshape(V * V, D2)
    out = _sc_gather(xp2d, ptab)
    return out.reshape(R, C, D)


# speedup vs baseline: 3.0922x; 1.0041x over previous
import functools

import jax
import jax.numpy as jnp
from jax import lax
from jax.experimental import pallas as pl
from jax.experimental.pallas import tpu as pltpu
from jax.experimental.pallas import tpu_sc as plsc

R, C = 16384, 200
V, D = 100, 36
D2 = 2 * D
NP = R * C // 2
G = 128
NG = NP // G
NC, NS = 2, 16
NW = NC * NS
GPW = NG // NW
K = 4
NCHUNK = GPW // K


def _sc_gather(xp2d, ptab):
    mesh = plsc.VectorSubcoreMesh(core_axis_name="c", subcore_axis_name="s")

    @functools.partial(
        pl.kernel,
        mesh=mesh,
        out_type=jax.ShapeDtypeStruct((NG, G, D2), jnp.float32),
        scratch_types=[
            pltpu.VMEM((3, K, G), jnp.int32),
            pltpu.VMEM((2, K, G, D2), jnp.float32),
            pltpu.SemaphoreType.DMA,
            pltpu.SemaphoreType.DMA,
            pltpu.SemaphoreType.DMA,
        ],
        compiler_params=pltpu.CompilerParams(use_tc_tiling_on_sc=False),
    )
    def k(xp_hbm, p_hbm, out_hbm, idx_v, rows_v, isem, gsem, ssem):
        wid = lax.axis_index("s") * NC + lax.axis_index("c")
        base = wid * GPW
        last = NCHUNK - 1

        def fire_idx(b, c):
            pltpu.async_copy(xp_hbm.at[pl.ds(base + c * K, K)], idx_v.at[b], isem)

        def wait_idx(b, c):
            pltpu.make_async_copy(
                xp_hbm.at[pl.ds(base + c * K, K)], idx_v.at[b], isem
            ).wait()

        def fire_gathers(bi, br):
            for j in range(K):
                pltpu.async_copy(p_hbm.at[idx_v.at[bi, j]], rows_v.at[br, j], gsem)

        def wait_gathers(br, c):
            pltpu.make_async_copy(
                out_hbm.at[pl.ds(base + c * K, K)], rows_v.at[br], gsem
            ).wait()

        def fire_scatter(br, c):
            pltpu.async_copy(rows_v.at[br], out_hbm.at[pl.ds(base + c * K, K)], ssem)

        def wait_scatter(br, c):
            pltpu.make_async_copy(
                rows_v.at[br], out_hbm.at[pl.ds(base + c * K, K)], ssem
            ).wait()

        fire_idx(0, 0)
        fire_idx(1, 1)
        wait_idx(0, 0)
        fire_gathers(0, 0)

        def body(i, carry):
            b3 = i % 3
            br = i % 2
            brp = (i - 1) % 2
            wait_idx(b3, i)

            @pl.when(i >= 2)
            def _():
                wait_scatter(br, i - 2)

            fire_gathers(b3, br)

            @pl.when(i < last)
            def _():
                fire_idx((i + 1) % 3, i + 1)

            wait_gathers(brp, i - 1)
            fire_scatter(brp, i - 1)
            return carry

        lax.fori_loop(1, NCHUNK, body, 0)

        wait_gathers(last % 2, last)
        fire_scatter(last % 2, last)
        wait_scatter((last - 1) % 2, last - 1)
        wait_scatter(last % 2, last)

    return k(xp2d, ptab)


def kernel(x, wte):
    xr = x.reshape(NP, 2).astype(jnp.int32)
    xp2d = (xr[:, 0] * V + xr[:, 1]).reshape(NG, G)
    ptab = jnp.concatenate(
        [
            jnp.broadcast_to(wte[:, None, :], (V, V, D)),
            jnp.broadcast_to(wte[None, :, :], (V, V, D)),
        ],
        axis=-1,
    ).reshape(V * V, D2)
    out = _sc_gather(xp2d, ptab)
    return out.reshape(R, C, D)
